# baseline (speedup 1.0000x reference)
import functools

import jax
import jax.numpy as jnp
from jax import lax
from jax.experimental import pallas as pl
from jax.experimental.pallas import tpu as pltpu
from jax.experimental.pallas import tpu_sc as plsc

SOFTMAX_SCALE = 0.08838834764831845
B, SQ, H, D = 8, 8, 16, 128
SKV, K = 8192, 2048
ROWS = B * SQ * K
ROWS_PER_B = SQ * K

NC, NS, L = 2, 16, 16
NW = NC * NS
CHUNK = 128

NSLAB = 4
SLAB_ROWS = ROWS // NSLAB
SLAB_PAIRS = (B * SQ) // NSLAB


def _sc_gather_slab(slab, idx_slab, kvp):
    rpt = SLAB_ROWS // NW
    nch = rpt // CHUNK
    mesh = plsc.VectorSubcoreMesh(core_axis_name="c", subcore_axis_name="s")

    @functools.partial(
        pl.kernel,
        out_type=jax.ShapeDtypeStruct((SLAB_ROWS, D), jnp.int32),
        mesh=mesh,
        scratch_types=[
            pltpu.VMEM((rpt,), jnp.int32),
            pltpu.VMEM((CHUNK, D), jnp.int32),
            pltpu.VMEM((CHUNK, D), jnp.int32),
            pltpu.SemaphoreType.DMA,
            pltpu.SemaphoreType.DMA,
        ],
    )
    def gather_kernel(idx_hbm, kv_hbm, out_hbm, idx_v, buf0, buf1, sem0, sem1):
        wid = lax.axis_index("s") * NC + lax.axis_index("c")
        tbase = wid * rpt
        pltpu.sync_copy(idx_hbm.at[pl.ds(tbase, rpt)], idx_v)
        badd = ((slab * SLAB_ROWS + tbase) // ROWS_PER_B) * SKV

        def add_body(i, carry):
            off = pl.multiple_of(i * L, L)
            idx_v[pl.ds(off, L)] = idx_v[pl.ds(off, L)] + badd
            return carry

        lax.fori_loop(0, rpt // L, add_body, 0)

        def start_gather(c, buf, sem):
            src = kv_hbm.at[idx_v.at[pl.ds(pl.multiple_of(c * CHUNK, CHUNK), CHUNK)]]
            pltpu.make_async_copy(src, buf, sem).start()

        def wait_gather(c, buf, sem):
            src = kv_hbm.at[idx_v.at[pl.ds(pl.multiple_of(c * CHUNK, CHUNK), CHUNK)]]
            pltpu.make_async_copy(src, buf, sem).wait()

        def writeback(c, buf):
            row = pl.multiple_of(tbase + c * CHUNK, CHUNK)
            pltpu.sync_copy(buf, out_hbm.at[pl.ds(row, CHUNK)])

        start_gather(0, buf0, sem0)

        def loop_body(i, carry):
            a = i * 2
            start_gather(a + 1, buf1, sem1)
            wait_gather(a, buf0, sem0)
            writeback(a, buf0)

            @pl.when(a + 2 < nch)
            def _():
                start_gather(a + 2, buf0, sem0)

            wait_gather(a + 1, buf1, sem1)
            writeback(a + 1, buf1)
            return carry

        lax.fori_loop(0, nch // 2, loop_body, 0)

    return gather_kernel(idx_slab, kvp)


PP = 1


def _attn_body(q_ref, kv_ref, sink_ref, o_ref):
    sink = sink_ref[...]
    for p_i in range(PP):
        q = q_ref[p_i]
        w = kv_ref[p_i]
        kf = lax.bitcast_convert_type(w << 16, jnp.float32)
        vf = lax.bitcast_convert_type(w & jnp.int32(-65536), jnp.float32)
        kb = kf.astype(jnp.bfloat16)
        vb = vf.astype(jnp.bfloat16)
        logits = lax.dot_general(
            q, kb, (((1,), (1,)), ((), ())), preferred_element_type=jnp.float32
        ) * SOFTMAX_SCALE
        m = jnp.maximum(jnp.max(logits, axis=1, keepdims=True), sink)
        e = jnp.exp(logits - m)
        denom = jnp.sum(e, axis=1, keepdims=True) + jnp.exp(sink - m)
        p = (e * (1.0 / denom)).astype(jnp.bfloat16)
        o_ref[p_i] = lax.dot_general(
            p, vb, (((1,), (0,)), ((), ())), preferred_element_type=jnp.float32
        )


def _tc_attention(q3, kv_sel3, sink_col):
    n = q3.shape[0]
    return pl.pallas_call(
        _attn_body,
        grid=(n // PP,),
        in_specs=[
            pl.BlockSpec((PP, H, D), lambda i: (i, 0, 0)),
            pl.BlockSpec((PP, K, D), lambda i: (i, 0, 0)),
            pl.BlockSpec((H, 1), lambda i: (0, 0)),
        ],
        out_specs=pl.BlockSpec((PP, H, D), lambda i: (i, 0, 0)),
        out_shape=jax.ShapeDtypeStruct((n, H, D), jnp.float32),
        compiler_params=pltpu.CompilerParams(
            dimension_semantics=("arbitrary",),
        ),
    )(q3, kv_sel3, sink_col)


def _pack_kv(kv):
    bits = lax.bitcast_convert_type(kv, jnp.uint32)
    rne = bits + (jnp.uint32(0x7FFF) + ((bits >> 16) & jnp.uint32(1)))
    kbits = rne[..., :D] >> 16
    vbits = rne[..., D:] & jnp.uint32(0xFFFF0000)
    return lax.bitcast_convert_type(kbits | vbits, jnp.int32).reshape(-1, D)


BATCHES_PER_SLAB = B // NSLAB


def kernel(q, kv, attn_sink, topk_idxs):
    idx_flat = topk_idxs.reshape(NSLAB, SLAB_ROWS)
    q4 = q.astype(jnp.bfloat16).reshape(NSLAB, SLAB_PAIRS, H, D)
    sink_col = attn_sink.reshape(H, 1)

    def pack_slab(s):
        return _pack_kv(kv[s * BATCHES_PER_SLAB:(s + 1) * BATCHES_PER_SLAB])

    kv_sels = [_sc_gather_slab(0, idx_flat[0], pack_slab(0))]
    outs = []
    for s in range(NSLAB):
        if s + 1 < NSLAB:
            kv_sels.append(_sc_gather_slab(0, idx_flat[s + 1], pack_slab(s + 1)))
        outs.append(
            _tc_attention(q4[s], kv_sels[s].reshape(SLAB_PAIRS, K, D), sink_col)
        )
    return jnp.stack(outs).reshape(B, SQ, H, D)


# speedup vs baseline: 1.1976x; 1.1976x over previous
import functools

import jax
import jax.numpy as jnp
from jax import lax
from jax.experimental import pallas as pl
from jax.experimental.pallas import tpu as pltpu
from jax.experimental.pallas import tpu_sc as plsc

SOFTMAX_SCALE = 0.08838834764831845
B, SQ, H, D = 8, 8, 16, 128
SKV, K = 8192, 2048
ROWS = B * SQ * K
ROWS_PER_B = SQ * K

NC, NS, L = 2, 16, 16
NW = NC * NS
CHUNK = 128

NSLAB = 4
SLAB_ROWS = ROWS // NSLAB
SLAB_PAIRS = (B * SQ) // NSLAB


def _sc_gather_slab(slab, idx_slab, kvp):
    rpt = SLAB_ROWS // NW
    nch = rpt // CHUNK
    mesh = plsc.VectorSubcoreMesh(core_axis_name="c", subcore_axis_name="s")

    @functools.partial(
        pl.kernel,
        out_type=jax.ShapeDtypeStruct((SLAB_ROWS, D), jnp.int32),
        mesh=mesh,
        scratch_types=[
            pltpu.VMEM((rpt,), jnp.int32),
            pltpu.VMEM((CHUNK, D), jnp.int32),
            pltpu.VMEM((CHUNK, D), jnp.int32),
            pltpu.SemaphoreType.DMA,
            pltpu.SemaphoreType.DMA,
        ],
    )
    def gather_kernel(idx_hbm, kv_hbm, out_hbm, idx_v, buf0, buf1, sem0, sem1):
        wid = lax.axis_index("s") * NC + lax.axis_index("c")
        tbase = wid * rpt
        pltpu.sync_copy(idx_hbm.at[pl.ds(tbase, rpt)], idx_v)
        badd = ((slab * SLAB_ROWS + tbase) // ROWS_PER_B) * SKV

        def add_body(i, carry):
            off = pl.multiple_of(i * L, L)
            idx_v[pl.ds(off, L)] = idx_v[pl.ds(off, L)] + badd
            return carry

        lax.fori_loop(0, rpt // L, add_body, 0)

        def start_gather(c, buf, sem):
            src = kv_hbm.at[idx_v.at[pl.ds(pl.multiple_of(c * CHUNK, CHUNK), CHUNK)]]
            pltpu.make_async_copy(src, buf, sem).start()

        def wait_gather(c, buf, sem):
            src = kv_hbm.at[idx_v.at[pl.ds(pl.multiple_of(c * CHUNK, CHUNK), CHUNK)]]
            pltpu.make_async_copy(src, buf, sem).wait()

        def writeback(c, buf):
            row = pl.multiple_of(tbase + c * CHUNK, CHUNK)
            pltpu.sync_copy(buf, out_hbm.at[pl.ds(row, CHUNK)])

        start_gather(0, buf0, sem0)

        def loop_body(i, carry):
            a = i * 2
            start_gather(a + 1, buf1, sem1)
            wait_gather(a, buf0, sem0)
            writeback(a, buf0)

            @pl.when(a + 2 < nch)
            def _():
                start_gather(a + 2, buf0, sem0)

            wait_gather(a + 1, buf1, sem1)
            writeback(a + 1, buf1)
            return carry

        lax.fori_loop(0, nch // 2, loop_body, 0)

    return gather_kernel(idx_slab, kvp)


PP = 2


def _attn_body(q_ref, kv_ref, sink_ref, o_ref):
    sink = sink_ref[...]
    for p_i in range(PP):
        q = q_ref[p_i]
        w = kv_ref[p_i]
        kf = lax.bitcast_convert_type(w << 16, jnp.float32)
        vf = lax.bitcast_convert_type(w & jnp.int32(-65536), jnp.float32)
        kb = kf.astype(jnp.bfloat16)
        vb = vf.astype(jnp.bfloat16)
        logits = lax.dot_general(
            q, kb, (((1,), (1,)), ((), ())), preferred_element_type=jnp.float32
        ) * SOFTMAX_SCALE
        m = jnp.maximum(jnp.max(logits, axis=1, keepdims=True), sink)
        e = jnp.exp(logits - m)
        denom = jnp.sum(e, axis=1, keepdims=True) + jnp.exp(sink - m)
        p = (e * (1.0 / denom)).astype(jnp.bfloat16)
        o_ref[p_i] = lax.dot_general(
            p, vb, (((1,), (0,)), ((), ())), preferred_element_type=jnp.float32
        )


def _tc_attention(q3, kv_sel3, sink_col):
    n = q3.shape[0]
    return pl.pallas_call(
        _attn_body,
        grid=(n // PP,),
        in_specs=[
            pl.BlockSpec((PP, H, D), lambda i: (i, 0, 0)),
            pl.BlockSpec((PP, K, D), lambda i: (i, 0, 0)),
            pl.BlockSpec((H, 1), lambda i: (0, 0)),
        ],
        out_specs=pl.BlockSpec((PP, H, D), lambda i: (i, 0, 0)),
        out_shape=jax.ShapeDtypeStruct((n, H, D), jnp.float32),
        compiler_params=pltpu.CompilerParams(
            dimension_semantics=("arbitrary",),
        ),
    )(q3, kv_sel3, sink_col)


def _pack_kv(kv):
    kvb = kv.astype(jnp.bfloat16)
    k16 = lax.bitcast_convert_type(kvb[..., :D], jnp.uint16).astype(jnp.uint32)
    v16 = lax.bitcast_convert_type(kvb[..., D:], jnp.uint16).astype(jnp.uint32)
    kvp = lax.bitcast_convert_type(k16 | (v16 << 16), jnp.int32)
    return kvp.reshape(-1, D)


BATCHES_PER_SLAB = B // NSLAB


def kernel(q, kv, attn_sink, topk_idxs):
    idx_flat = topk_idxs.reshape(NSLAB, SLAB_ROWS)
    q4 = q.astype(jnp.bfloat16).reshape(NSLAB, SLAB_PAIRS, H, D)
    sink_col = attn_sink.reshape(H, 1)

    def pack_slab(s):
        return _pack_kv(kv[s * BATCHES_PER_SLAB:(s + 1) * BATCHES_PER_SLAB])

    kv_sels = [_sc_gather_slab(0, idx_flat[0], pack_slab(0))]
    outs = []
    for s in range(NSLAB):
        if s + 1 < NSLAB:
            kv_sels.append(_sc_gather_slab(0, idx_flat[s + 1], pack_slab(s + 1)))
        outs.append(
            _tc_attention(q4[s], kv_sels[s].reshape(SLAB_PAIRS, K, D), sink_col)
        )
    return jnp.stack(outs).reshape(B, SQ, H, D)
